# baseline (speedup 1.0000x reference)
import functools

import jax
import jax.numpy as jnp
from jax import lax
from jax.experimental import pallas as pl
from jax.experimental.pallas import tpu as pltpu
from jax.experimental.pallas import tpu_sc as plsc

NC = 2
NS = 16
NT = NC * NS
CHUNK = 128


def _cdiv(a, b):
    return (a + b - 1) // b


def _make_degree(k, hrows):
    rq = hrows // CHUNK
    zpt = hrows // NS
    mesh = plsc.VectorSubcoreMesh(core_axis_name="c", subcore_axis_name="s")

    @functools.partial(
        pl.kernel,
        out_type=jax.ShapeDtypeStruct((NC, hrows, CHUNK), jnp.float32),
        mesh=mesh,
        scratch_types=[
            pltpu.VMEM((k, CHUNK), jnp.int32),
            pltpu.VMEM((hrows, CHUNK), jnp.float32),
            pltpu.VMEM((rq, CHUNK), jnp.int32),
            pltpu.VMEM_SHARED((hrows, CHUNK), jnp.float32),
            pltpu.SemaphoreType.DMA,
        ],
        compiler_params=pltpu.CompilerParams(needs_layout_passes=False),
    )
    def deg_kernel(dst_hbm, zeros_hbm, rowids_hbm, out_hbm,
                   dst_v, hist_v, rowid_v, acc, sem):
        c = lax.axis_index("c")
        s = lax.axis_index("s")
        w = c * NS + s
        pltpu.sync_copy(zeros_hbm.at[pl.ds(0, zpt)], acc.at[pl.ds(s * zpt, zpt)])
        pltpu.sync_copy(zeros_hbm, hist_v)
        pltpu.sync_copy(rowids_hbm, rowid_v)
        pltpu.async_copy(dst_hbm.at[w], dst_v, sem).wait()
        plsc.subcore_barrier()

        hn = hrows * 16
        iota = lax.broadcasted_iota(jnp.int32, (16,), 0)
        lane8 = jnp.bitwise_and(iota, 7)
        mask_lo = iota < 8
        mask_hi = iota >= 8
        ones_v = jnp.full((16,), 1.0, jnp.float32)

        @pl.loop(0, k)
        def _(j):
            for i in range(CHUNK // 16):
                dst16 = dst_v[j, pl.ds(i * 16, 16)]
                flat = lane8 * hn + dst16
                row = lax.shift_right_logical(flat, 7)
                lane = jnp.bitwise_and(flat, 127)
                plsc.addupdate_scatter(hist_v, [row, lane], ones_v,
                                       mask=mask_lo)
                plsc.addupdate_scatter(hist_v, [row, lane], ones_v,
                                       mask=mask_hi)

        for q in range(rq):
            pltpu.sync_copy(hist_v.at[pl.ds(q * CHUNK, CHUNK)],
                            acc.at[rowid_v.at[q]], add=True)

        plsc.subcore_barrier()
        pltpu.sync_copy(acc.at[pl.ds(s * zpt, zpt)],
                        out_hbm.at[c, pl.ds(s * zpt, zpt)])

    return deg_kernel


def _make_aggregate(out_rows, d, k, acc_rows):
    zpt = acc_rows // NS
    rpt = out_rows // NS
    mesh = plsc.VectorSubcoreMesh(core_axis_name="c", subcore_axis_name="s")

    @functools.partial(
        pl.kernel,
        out_type=jax.ShapeDtypeStruct((NC, out_rows, d), jnp.float32),
        mesh=mesh,
        scratch_types=[
            pltpu.VMEM((k, CHUNK), jnp.int32),
            pltpu.VMEM((k, CHUNK), jnp.int32),
            pltpu.VMEM((CHUNK, d), jnp.float32),
            pltpu.VMEM_SHARED((acc_rows, d), jnp.float32),
            pltpu.SemaphoreType.DMA,
        ],
    )
    def agg_kernel(hp_hbm, src_hbm, dst_hbm, zeros_hbm, out_hbm,
                   src_v, dst_v, rows_v, acc, sem):
        c = lax.axis_index("c")
        s = lax.axis_index("s")
        w = c * NS + s
        pltpu.sync_copy(zeros_hbm, acc.at[pl.ds(s * zpt, zpt)])
        pltpu.async_copy(src_hbm.at[w], src_v, sem).wait()
        pltpu.async_copy(dst_hbm.at[w], dst_v, sem).wait()
        plsc.subcore_barrier()

        @pl.loop(0, k)
        def _(j):
            pltpu.async_copy(hp_hbm.at[src_v.at[j]], rows_v, sem).wait()
            pltpu.sync_copy(rows_v, acc.at[dst_v.at[j]], add=True)

        plsc.subcore_barrier()
        pltpu.sync_copy(acc.at[pl.ds(s * rpt, rpt)],
                        out_hbm.at[c, pl.ds(s * rpt, rpt)])

    return agg_kernel


_ROWS = 1000


def _deg_dis(dc_ref):
    deg = dc_ref[:, :1]
    return deg, lax.rsqrt(deg)


def _tc_deg_finalize(h0, h1):
    hrows, _ = h0.shape
    grows = hrows // 8
    nodes = grows * CHUNK
    blk = 1024

    def body(h0_ref, h1_ref, o_ref):
        i = pl.program_id(0)
        flat = h0_ref[...] + h1_ref[...]
        byn = flat[0:grows, :]
        for g in range(1, 8):
            byn = byn + flat[g * grows:(g + 1) * grows, :]
        rows = lax.broadcasted_iota(jnp.int32, (blk, grows), 0)
        cols = lax.broadcasted_iota(jnp.int32, (blk, grows), 1)
        P = (cols == (blk // CHUNK) * i + rows // CHUNK).astype(jnp.float32)
        Y = jnp.dot(P, byn, preferred_element_type=jnp.float32,
                    precision=lax.Precision.HIGHEST)
        rj = lax.broadcasted_iota(jnp.int32, (blk, CHUNK), 0)
        lj = lax.broadcasted_iota(jnp.int32, (blk, CHUNK), 1)
        M = (lj == jnp.bitwise_and(rj, CHUNK - 1)).astype(jnp.float32)
        deg = 1.0 + jnp.sum(Y * M, axis=1, keepdims=True)
        o_ref[...] = deg + jnp.zeros((1, 16), jnp.float32)

    return pl.pallas_call(
        body,
        grid=(nodes // blk,),
        in_specs=[
            pl.BlockSpec((hrows, CHUNK), lambda i: (0, 0)),
            pl.BlockSpec((hrows, CHUNK), lambda i: (0, 0)),
        ],
        out_specs=pl.BlockSpec((blk, 16), lambda i: (i, 0)),
        out_shape=jax.ShapeDtypeStruct((nodes, 16), jnp.float32),
    )(h0, h1)


def _tc_prescale(x, w1, dc):
    n, din = x.shape
    d = w1.shape[1]

    def body(x_ref, w_ref, dc_ref, o_ref):
        _, dis = _deg_dis(dc_ref)
        h = jnp.dot(x_ref[...], w_ref[...], preferred_element_type=jnp.float32)
        o_ref[...] = h * dis

    return pl.pallas_call(
        body,
        grid=(n // _ROWS,),
        in_specs=[
            pl.BlockSpec((_ROWS, din), lambda i: (i, 0)),
            pl.BlockSpec((din, d), lambda i: (0, 0)),
            pl.BlockSpec((_ROWS, 16), lambda i: (i, 0)),
        ],
        out_specs=pl.BlockSpec((_ROWS, d), lambda i: (i, 0)),
        out_shape=jax.ShapeDtypeStruct((n, d), jnp.float32),
    )(x, w1, dc)


def _tc_mid(a0, a1, hp1, dc, w2, b1):
    n, d = hp1.shape
    dout = w2.shape[1]

    def body(a0_ref, a1_ref, hp_ref, dc_ref, w_ref, b_ref, o_ref):
        _, dis = _deg_dis(dc_ref)
        o1 = dis * (a0_ref[...] + a1_ref[...] + hp_ref[...]) + b_ref[...]
        o1 = jnp.maximum(o1, 0.0)
        h2 = jnp.dot(o1, w_ref[...], preferred_element_type=jnp.float32)
        o_ref[...] = h2 * dis

    full = pl.BlockSpec((_ROWS, d), lambda i: (i, 0))
    return pl.pallas_call(
        body,
        grid=(n // _ROWS,),
        in_specs=[
            full, full, full,
            pl.BlockSpec((_ROWS, 16), lambda i: (i, 0)),
            pl.BlockSpec((d, dout), lambda i: (0, 0)),
            pl.BlockSpec((1, d), lambda i: (0, 0)),
        ],
        out_specs=pl.BlockSpec((_ROWS, dout), lambda i: (i, 0)),
        out_shape=jax.ShapeDtypeStruct((n, dout), jnp.float32),
    )(a0, a1, hp1, dc, w2, b1)


def _tc_final(a0, a1, hp2, dc, b2):
    n, d = hp2.shape

    def body(a0_ref, a1_ref, hp_ref, dc_ref, b_ref, o_ref):
        _, dis = _deg_dis(dc_ref)
        o2 = dis * (a0_ref[...] + a1_ref[...] + hp_ref[...]) + b_ref[...]
        o2 = jnp.maximum(o2, 0.0)
        m = jnp.max(o2, axis=1, keepdims=True)
        shifted = o2 - m
        lse = jnp.log(jnp.sum(jnp.exp(shifted), axis=1, keepdims=True))
        o_ref[...] = shifted - lse

    full = pl.BlockSpec((_ROWS, d), lambda i: (i, 0))
    return pl.pallas_call(
        body,
        grid=(n // _ROWS,),
        in_specs=[
            full, full, full,
            pl.BlockSpec((_ROWS, 16), lambda i: (i, 0)),
            pl.BlockSpec((1, d), lambda i: (0, 0)),
        ],
        out_specs=pl.BlockSpec((_ROWS, d), lambda i: (i, 0)),
        out_shape=jax.ShapeDtypeStruct((n, d), jnp.float32),
    )(a0, a1, hp2, dc, b2)


def kernel(x, adj_t, W1, b1, W2, b2):
    n, din = x.shape
    e = adj_t.shape[1]
    d = W1.shape[1]

    k = _cdiv(e, NT * CHUNK)
    e_pad = NT * k * CHUNK
    out_rows = _cdiv(n, 128) * 128
    acc_rows = out_rows + 128

    src = adj_t[0]
    dst = adj_t[1]
    pad = e_pad - e
    srcp = jnp.concatenate([src, jnp.zeros((pad,), src.dtype)])
    dstp = jnp.concatenate([dst, jnp.full((pad,), n, dst.dtype)])
    src3 = srcp.reshape(NT, k, CHUNK)
    dst3 = dstp.reshape(NT, k, CHUNK)

    hrows = _cdiv((n + 1) * 8, CHUNK * CHUNK) * CHUNK

    zeros_hist = jnp.zeros((hrows, CHUNK), jnp.float32)
    rowids = jnp.arange(hrows, dtype=jnp.int32).reshape(hrows // CHUNK, CHUNK)
    zeros_d = jnp.zeros((acc_rows // NS, d), jnp.float32)

    degp = _make_degree(k, hrows)(dst3, zeros_hist, rowids)
    dc = _tc_deg_finalize(degp[0], degp[1])[:n]

    agg = _make_aggregate(out_rows, d, k, acc_rows)
    hp1 = _tc_prescale(x, W1, dc)
    agg1 = agg(hp1, src3, dst3, zeros_d)
    hp2 = _tc_mid(agg1[0, :n], agg1[1, :n], hp1, dc, W2, b1.reshape(1, d))
    agg2 = agg(hp2, src3, dst3, zeros_d)
    return _tc_final(agg2[0, :n], agg2[1, :n], hp2, dc, b2.reshape(1, d))
